# baseline (speedup 1.0000x reference)
import functools

import jax
import jax.numpy as jnp
from jax import lax
from jax.experimental import pallas as pl
from jax.experimental.pallas import tpu as pltpu
from jax.experimental.pallas import tpu_sc as plsc

_NV = 8
_GB = 16
_SCH = 8


def _vote_body(x_ref, i_ref):
    block = x_ref[...]
    gb, nv, n = block.shape
    t = pl.program_id(0)
    rowmax = jnp.max(block, axis=2)
    m = jnp.max(rowmax, axis=1, keepdims=True)
    ismax = rowmax == m
    counts = jnp.sum(ismax.astype(jnp.int32), axis=1)
    viota = jax.lax.broadcasted_iota(jnp.int32, (gb, nv), 1)
    votes_fast = jnp.min(jnp.where(ismax, viota, jnp.int32(nv)), axis=1)

    def _tie_vote(g):
        def _inner():
            vgrid = jax.lax.broadcasted_iota(jnp.int32, (nv, n), 0)
            fgrid = jax.lax.broadcasted_iota(jnp.int32, (nv, n), 1)
            keys = jnp.where(block[g] == m[g, 0], fgrid * nv + vgrid,
                             jnp.int32(2**31 - 1))
            return jnp.min(keys) % nv
        return _inner

    giota = jax.lax.broadcasted_iota(jnp.int32, (gb,), 0)
    votes = votes_fast
    for g in range(gb):
        vote = jax.lax.cond(counts[g] > 1, _tie_vote(g),
                            lambda vf=votes_fast[g]: vf)
        votes = jnp.where(giota == g, vote, votes)
    i_ref[0, 0, :] = (t * gb + giota) * _NV + votes


def _make_votes(b, n, interpret=False):
    return pl.pallas_call(
        _vote_body,
        grid=(b // _GB,),
        in_specs=[pl.BlockSpec((_GB, _NV, n), lambda g: (g, 0, 0))],
        out_specs=pl.BlockSpec((1, 1, _GB), lambda g: (g, 0, 0)),
        out_shape=jax.ShapeDtypeStruct((b // _GB, 1, _GB), jnp.int32),
        interpret=interpret,
    )


_RPG = 1
_NBUF = 2


def _make_sc_gather(b, n):
    nc = 2
    ns = 16
    nw = nc * ns
    rpw = b // nw
    ng = rpw // _RPG
    mesh = plsc.VectorSubcoreMesh(core_axis_name="c", subcore_axis_name="s")

    @functools.partial(
        pl.kernel,
        mesh=mesh,
        out_type=jax.ShapeDtypeStruct((b, n), jnp.float32),
        scratch_types=[
            pltpu.VMEM((ng, _RPG), jnp.int32),
            pltpu.VMEM((_NBUF, _RPG, n), jnp.float32),
            pltpu.SemaphoreType.DMA((_NBUF,)),
        ],
    )
    def _sc(x_hbm, ids_hbm, out_hbm, idx_v, rows_v, sems):
        wid = lax.axis_index("s") * nc + lax.axis_index("c")
        pltpu.sync_copy(ids_hbm.at[wid], idx_v)

        def _start(h):
            return pltpu.async_copy(x_hbm.at[idx_v.at[h]],
                                    rows_v.at[h % _NBUF], sems.at[h % _NBUF])

        copies = [None] * ng
        for h in range(min(_NBUF, ng)):
            copies[h] = _start(h)
        for h in range(ng):
            copies[h].wait()
            pltpu.sync_copy(rows_v.at[h % _NBUF],
                            out_hbm.at[pl.ds(wid * rpw + h * _RPG, _RPG)])
            if h + _NBUF < ng:
                copies[h + _NBUF] = _start(h + _NBUF)

    return _sc


def kernel(x):
    b = x.shape[0] // _NV
    xr = x.reshape(b, _NV, -1)
    n = xr.shape[-1]
    nw = 32
    ids = _make_votes(b, n)(xr).reshape(nw, b // (nw * _RPG), _RPG)
    out = _make_sc_gather(b, n)(x, ids)
    return out


# speedup vs baseline: 1.0109x; 1.0109x over previous
import functools

import jax
import jax.numpy as jnp
from jax import lax
from jax.experimental import pallas as pl
from jax.experimental.pallas import tpu as pltpu
from jax.experimental.pallas import tpu_sc as plsc

_NV = 8
_GB = 8
_SCH = 8


def _vote_body(x_ref, i_ref):
    block = x_ref[...]
    gb, nv, n = block.shape
    t = pl.program_id(0)
    rowmax = jnp.max(block, axis=2)
    m = jnp.max(rowmax, axis=1, keepdims=True)
    ismax = rowmax == m
    counts = jnp.sum(ismax.astype(jnp.int32), axis=1)
    viota = jax.lax.broadcasted_iota(jnp.int32, (gb, nv), 1)
    votes_fast = jnp.min(jnp.where(ismax, viota, jnp.int32(nv)), axis=1)

    def _tie_vote(g):
        def _inner():
            vgrid = jax.lax.broadcasted_iota(jnp.int32, (nv, n), 0)
            fgrid = jax.lax.broadcasted_iota(jnp.int32, (nv, n), 1)
            keys = jnp.where(block[g] == m[g, 0], fgrid * nv + vgrid,
                             jnp.int32(2**31 - 1))
            return jnp.min(keys) % nv
        return _inner

    giota = jax.lax.broadcasted_iota(jnp.int32, (gb,), 0)
    votes = votes_fast
    for g in range(gb):
        vote = jax.lax.cond(counts[g] > 1, _tie_vote(g),
                            lambda vf=votes_fast[g]: vf)
        votes = jnp.where(giota == g, vote, votes)
    i_ref[0, 0, :] = (t * gb + giota) * _NV + votes


def _make_votes(b, n, interpret=False):
    return pl.pallas_call(
        _vote_body,
        grid=(b // _GB,),
        in_specs=[pl.BlockSpec((_GB, _NV, n), lambda g: (g, 0, 0))],
        out_specs=pl.BlockSpec((1, 1, _GB), lambda g: (g, 0, 0)),
        out_shape=jax.ShapeDtypeStruct((b // _GB, 1, _GB), jnp.int32),
        interpret=interpret,
    )


_RPG = 1
_NBUF = 2


def _make_sc_gather(b, n):
    nc = 2
    ns = 16
    nw = nc * ns
    rpw = b // nw
    ng = rpw // _RPG
    mesh = plsc.VectorSubcoreMesh(core_axis_name="c", subcore_axis_name="s")

    @functools.partial(
        pl.kernel,
        mesh=mesh,
        out_type=jax.ShapeDtypeStruct((b, n), jnp.float32),
        scratch_types=[
            pltpu.VMEM((ng, _RPG), jnp.int32),
            pltpu.VMEM((_NBUF, _RPG, n), jnp.float32),
            pltpu.SemaphoreType.DMA((_NBUF,)),
        ],
    )
    def _sc(x_hbm, ids_hbm, out_hbm, idx_v, rows_v, sems):
        wid = lax.axis_index("s") * nc + lax.axis_index("c")
        pltpu.sync_copy(ids_hbm.at[wid], idx_v)

        def _start(h):
            return pltpu.async_copy(x_hbm.at[idx_v.at[h]],
                                    rows_v.at[h % _NBUF], sems.at[h % _NBUF])

        copies = [None] * ng
        for h in range(min(_NBUF, ng)):
            copies[h] = _start(h)
        for h in range(ng):
            copies[h].wait()
            pltpu.sync_copy(rows_v.at[h % _NBUF],
                            out_hbm.at[pl.ds(wid * rpw + h * _RPG, _RPG)])
            if h + _NBUF < ng:
                copies[h + _NBUF] = _start(h + _NBUF)

    return _sc


def kernel(x):
    b = x.shape[0] // _NV
    xr = x.reshape(b, _NV, -1)
    n = xr.shape[-1]
    nw = 32
    ids = _make_votes(b, n)(xr).reshape(nw, b // (nw * _RPG), _RPG)
    out = _make_sc_gather(b, n)(x, ids)
    return out


# speedup vs baseline: 1.0179x; 1.0069x over previous
import functools

import jax
import jax.numpy as jnp
from jax import lax
from jax.experimental import pallas as pl
from jax.experimental.pallas import tpu as pltpu
from jax.experimental.pallas import tpu_sc as plsc

_NV = 8
_GB = 8
_SCH = 8


def _vote_body(x_ref, i_ref):
    block = x_ref[...]
    gb, nv, n = block.shape
    t = pl.program_id(0)
    rowmax = jnp.max(block, axis=2)
    m = jnp.max(rowmax, axis=1, keepdims=True)
    ismax = rowmax == m
    counts = jnp.sum(ismax.astype(jnp.int32), axis=1)
    viota = jax.lax.broadcasted_iota(jnp.int32, (gb, nv), 1)
    votes_fast = jnp.min(jnp.where(ismax, viota, jnp.int32(nv)), axis=1)

    def _tie_vote(g):
        def _inner():
            vgrid = jax.lax.broadcasted_iota(jnp.int32, (nv, n), 0)
            fgrid = jax.lax.broadcasted_iota(jnp.int32, (nv, n), 1)
            keys = jnp.where(block[g] == m[g, 0], fgrid * nv + vgrid,
                             jnp.int32(2**31 - 1))
            return jnp.min(keys) % nv
        return _inner

    giota = jax.lax.broadcasted_iota(jnp.int32, (gb,), 0)
    votes = votes_fast
    for g in range(gb):
        vote = jax.lax.cond(counts[g] > 1, _tie_vote(g),
                            lambda vf=votes_fast[g]: vf)
        votes = jnp.where(giota == g, vote, votes)
    i_ref[0, 0, :] = (t * gb + giota) * _NV + votes


def _make_votes(b, n, interpret=False):
    return pl.pallas_call(
        _vote_body,
        grid=(b // _GB,),
        in_specs=[pl.BlockSpec((_GB, _NV, n), lambda g: (g, 0, 0))],
        out_specs=pl.BlockSpec((1, 1, _GB), lambda g: (g, 0, 0)),
        out_shape=jax.ShapeDtypeStruct((b // _GB, 1, _GB), jnp.int32),
        interpret=interpret,
    )


_RPG = 1
_NBUF = 3


def _make_sc_gather(b, n):
    nc = 2
    ns = 16
    nw = nc * ns
    rpw = b // nw
    ng = rpw // _RPG
    mesh = plsc.VectorSubcoreMesh(core_axis_name="c", subcore_axis_name="s")

    @functools.partial(
        pl.kernel,
        mesh=mesh,
        out_type=jax.ShapeDtypeStruct((b, n), jnp.float32),
        scratch_types=[
            pltpu.VMEM((ng, _RPG), jnp.int32),
            pltpu.VMEM((_NBUF, _RPG, n), jnp.float32),
            pltpu.SemaphoreType.DMA((_NBUF,)),
            pltpu.SemaphoreType.DMA((_NBUF,)),
        ],
    )
    def _sc(x_hbm, ids_hbm, out_hbm, idx_v, rows_v, semi, semo):
        wid = lax.axis_index("s") * nc + lax.axis_index("c")
        pltpu.sync_copy(ids_hbm.at[wid], idx_v)

        def _gather(h):
            return pltpu.async_copy(x_hbm.at[idx_v.at[h]],
                                    rows_v.at[h % _NBUF], semi.at[h % _NBUF])

        ci = [None] * ng
        co = [None] * ng
        for h in range(min(_NBUF, ng)):
            ci[h] = _gather(h)
        for h in range(ng):
            ci[h].wait()
            co[h] = pltpu.async_copy(
                rows_v.at[h % _NBUF],
                out_hbm.at[pl.ds(wid * rpw + h * _RPG, _RPG)],
                semo.at[h % _NBUF])
            if h + _NBUF < ng:
                co[h].wait()
                ci[h + _NBUF] = _gather(h + _NBUF)
        for h in range(ng):
            if h + _NBUF >= ng:
                co[h].wait()

    return _sc


def kernel(x):
    b = x.shape[0] // _NV
    xr = x.reshape(b, _NV, -1)
    n = xr.shape[-1]
    nw = 32
    ids = _make_votes(b, n)(xr).reshape(nw, b // (nw * _RPG), _RPG)
    out = _make_sc_gather(b, n)(x, ids)
    return out


# speedup vs baseline: 1.0192x; 1.0013x over previous
import functools

import jax
import jax.numpy as jnp
from jax import lax
from jax.experimental import pallas as pl
from jax.experimental.pallas import tpu as pltpu
from jax.experimental.pallas import tpu_sc as plsc

_NV = 8
_GB = 8


def _vote_body(x_ref, i_ref):
    block = x_ref[...]
    gb, nv, n = block.shape
    t = pl.program_id(0)
    rowmax = jnp.max(block, axis=2)
    m = jnp.max(rowmax, axis=1, keepdims=True)
    ismax = rowmax == m
    counts = jnp.sum(ismax.astype(jnp.int32), axis=1)
    viota = jax.lax.broadcasted_iota(jnp.int32, (gb, nv), 1)
    votes_fast = jnp.min(jnp.where(ismax, viota, jnp.int32(nv)), axis=1)

    def _tie_vote(g):
        def _inner():
            vgrid = jax.lax.broadcasted_iota(jnp.int32, (nv, n), 0)
            fgrid = jax.lax.broadcasted_iota(jnp.int32, (nv, n), 1)
            keys = jnp.where(block[g] == m[g, 0], fgrid * nv + vgrid,
                             jnp.int32(2**31 - 1))
            return jnp.min(keys) % nv
        return _inner

    giota = jax.lax.broadcasted_iota(jnp.int32, (gb,), 0)
    votes = votes_fast
    for g in range(gb):
        vote = jax.lax.cond(counts[g] > 1, _tie_vote(g),
                            lambda vf=votes_fast[g]: vf)
        votes = jnp.where(giota == g, vote, votes)
    i_ref[0, 0, :] = (t * gb + giota) * _NV + votes


def _make_votes(b, n, interpret=False):
    return pl.pallas_call(
        _vote_body,
        grid=(b // _GB,),
        in_specs=[pl.BlockSpec((_GB, _NV, n), lambda g: (g, 0, 0))],
        out_specs=pl.BlockSpec((1, 1, _GB), lambda g: (g, 0, 0)),
        out_shape=jax.ShapeDtypeStruct((b // _GB, 1, _GB), jnp.int32),
        interpret=interpret,
    )


_RPG = 1
_NBUF = 3


def _make_sc_gather(b, n):
    nc = 2
    ns = 16
    nw = nc * ns
    rpw = b // nw
    ng = rpw // _RPG
    mesh = plsc.VectorSubcoreMesh(core_axis_name="c", subcore_axis_name="s")

    @functools.partial(
        pl.kernel,
        mesh=mesh,
        out_type=jax.ShapeDtypeStruct((b, n), jnp.float32),
        scratch_types=[
            pltpu.VMEM((ng, _RPG), jnp.int32),
            pltpu.VMEM((_NBUF, _RPG, n), jnp.float32),
            pltpu.SemaphoreType.DMA((_NBUF,)),
            pltpu.SemaphoreType.DMA((_NBUF,)),
        ],
    )
    def _sc(x_hbm, ids_hbm, out_hbm, idx_v, rows_v, semi, semo):
        wid = lax.axis_index("s") * nc + lax.axis_index("c")
        pltpu.sync_copy(ids_hbm.at[wid], idx_v)

        def _gather(h):
            return pltpu.async_copy(x_hbm.at[idx_v.at[h]],
                                    rows_v.at[h % _NBUF], semi.at[h % _NBUF])

        ci = [None] * ng
        co = [None] * ng
        for h in range(min(_NBUF, ng)):
            ci[h] = _gather(h)
        for h in range(ng):
            ci[h].wait()
            co[h] = pltpu.async_copy(
                rows_v.at[h % _NBUF],
                out_hbm.at[pl.ds(wid * rpw + h * _RPG, _RPG)],
                semo.at[h % _NBUF])
            if h + _NBUF < ng:
                co[h].wait()
                ci[h + _NBUF] = _gather(h + _NBUF)
        for h in range(ng):
            if h + _NBUF >= ng:
                co[h].wait()

    return _sc


def kernel(x):
    b = x.shape[0] // _NV
    xr = x.reshape(b, _NV, -1)
    n = xr.shape[-1]
    nw = 32
    ids = _make_votes(b, n)(xr).reshape(nw, b // (nw * _RPG), _RPG)
    out = _make_sc_gather(b, n)(x, ids)
    return out
